# Initial kernel scaffold; baseline (speedup 1.0000x reference)
import jax
import jax.numpy as jnp
from jax.experimental import pallas as pl


def kernel(x, edge_index, batch, W1, b1, W2, b2, Wfc, bfc):
    raise NotImplementedError("write your pallas kernel here")


# speedup vs baseline: 7.2049x; 7.2049x over previous
import functools

import jax
import jax.numpy as jnp
from jax import lax
from jax.experimental import pallas as pl
from jax.experimental.pallas import tpu as pltpu
from jax.experimental.pallas import tpu_sc as plsc

N = 10000
E = 320000
HID = 128
NC_OUT = 64
NG = 64

NTILE = 16
CHUNK = 128
N_CHUNKS = 158
EPW = N_CHUNKS * CHUNK
E_PAD = NTILE * EPW
N_PAD = 10240
ROWS_PER_TILE = N_PAD // NTILE

BM = 2048
GRID = N_PAD // BM

_mesh = plsc.VectorSubcoreMesh(core_axis_name="c", subcore_axis_name="s",
                               num_cores=1, num_subcores=NTILE)
_sc_params = pltpu.CompilerParams(needs_layout_passes=False)


@functools.partial(
    pl.kernel,
    out_type=jax.ShapeDtypeStruct((NTILE, N_PAD), jnp.float32),
    mesh=_mesh,
    compiler_params=_sc_params,
    scratch_types=[
        pltpu.VMEM((CHUNK,), jnp.int32),
        pltpu.VMEM((N_PAD,), jnp.float32),
    ],
)
def _deg_kernel(dst_hbm, out_hbm, idx_v, hist_v):
    s = lax.axis_index("s")

    zv = jnp.zeros((16,), jnp.float32)
    pl.loop(0, N_PAD // 16)(
        lambda i: hist_v.__setitem__(pl.ds(i * 16, 16), zv))

    ones = jnp.ones((16,), jnp.float32)

    def step(k):
        base = s * EPW + k * CHUNK
        pltpu.sync_copy(dst_hbm.at[pl.ds(base, CHUNK)], idx_v)
        for j in range(CHUNK // 16):
            idx16 = idx_v[pl.ds(j * 16, 16)]
            plsc.addupdate_scatter(hist_v, [idx16], ones)

    pl.loop(0, N_CHUNKS)(step)
    pltpu.sync_copy(hist_v, out_hbm.at[s])


@functools.partial(
    pl.kernel,
    out_type=jax.ShapeDtypeStruct((N_PAD, HID), jnp.float32),
    mesh=_mesh,
    compiler_params=_sc_params,
    scratch_types=[
        pltpu.VMEM((CHUNK,), jnp.int32),
        pltpu.VMEM((CHUNK,), jnp.int32),
        pltpu.VMEM((CHUNK, HID), jnp.float32),
        pltpu.VMEM((CHUNK, HID), jnp.float32),
        pltpu.VMEM_SHARED((N_PAD, HID), jnp.float32),
        pltpu.SemaphoreType.DMA,
    ],
)
def _agg_kernel(src_hbm, dst_hbm, g_hbm, out_hbm,
                src_v, dst_v, rows_v, z_v, acc_sh, sem):
    s = lax.axis_index("s")

    zv = jnp.zeros((16,), jnp.float32)

    def zrow(i):
        for j in range(HID // 16):
            z_v[i, pl.ds(j * 16, 16)] = zv

    pl.loop(0, CHUNK)(zrow)

    row0 = s * ROWS_PER_TILE
    for r in range(0, ROWS_PER_TILE, CHUNK):
        nr = min(CHUNK, ROWS_PER_TILE - r)
        pltpu.sync_copy(z_v.at[pl.ds(0, nr)],
                        acc_sh.at[pl.ds(row0 + r, nr)])
    plsc.subcore_barrier()

    def step(k):
        base = s * EPW + k * CHUNK
        pltpu.sync_copy(src_hbm.at[pl.ds(base, CHUNK)], src_v)
        pltpu.sync_copy(dst_hbm.at[pl.ds(base, CHUNK)], dst_v)
        pltpu.async_copy(g_hbm.at[src_v], rows_v, sem).wait()
        pltpu.sync_copy(rows_v, acc_sh.at[dst_v], add=True)

    pl.loop(0, N_CHUNKS)(step)
    plsc.subcore_barrier()

    for r in range(0, ROWS_PER_TILE, CHUNK):
        nr = min(CHUNK, ROWS_PER_TILE - r)
        pltpu.sync_copy(acc_sh.at[pl.ds(row0 + r, nr)], z_v.at[pl.ds(0, nr)])
        pltpu.sync_copy(z_v.at[pl.ds(0, nr)],
                        out_hbm.at[pl.ds(row0 + r, nr)])


_DOT = dict(preferred_element_type=jnp.float32,
            precision=lax.Precision.HIGHEST)


def _tc1_body(degp_ref, x_ref, w1_ref, g_ref, dinv_ref):
    ones16 = jnp.ones((NTILE, 1), jnp.float32)
    deg = lax.dot_general(degp_ref[...], ones16,
                          (((0,), (0,)), ((), ())), **_DOT) + 1.0
    dinv = lax.rsqrt(deg)
    h = jnp.dot(x_ref[...], w1_ref[...], **_DOT)
    g_ref[...] = h * dinv
    dinv_ref[...] = jnp.broadcast_to(dinv, (BM, 16))


def _tc2_body(acc_ref, g1_ref, dinv_ref, b1_ref, w2_ref, g2_ref):
    dinv = dinv_ref[:, 0:1]
    out1 = (acc_ref[...] + g1_ref[...]) * dinv + b1_ref[...]
    out1 = jnp.maximum(out1, 0.0)
    g2_ref[...] = jnp.dot(out1, w2_ref[...], **_DOT) * dinv


def _tc3_body(acc_ref, g2_ref, dinv_ref, b2_ref, batch_ref,
              wfc_ref, bfc_ref, out_ref, sums_scr, cnt_scr):
    i = pl.program_id(0)

    @pl.when(i == 0)
    def _():
        sums_scr[...] = jnp.zeros_like(sums_scr)
        cnt_scr[...] = jnp.zeros_like(cnt_scr)

    dinv = dinv_ref[:, 0:1]
    out2 = (acc_ref[...] + g2_ref[...]) * dinv + b2_ref[...]
    out2 = jnp.maximum(out2, 0.0)

    gids = lax.broadcasted_iota(jnp.int32, (BM, NG), 1)
    onehot = (batch_ref[:, 0:1] == gids).astype(jnp.float32)
    sums_scr[...] += lax.dot_general(
        onehot, out2, (((0,), (0,)), ((), ())), **_DOT)
    cnt = jnp.sum(onehot, axis=0)
    cnt_scr[...] += jnp.broadcast_to(cnt[:, None], (NG, HID))

    @pl.when(i == GRID - 1)
    def _():
        pooled = sums_scr[...] / jnp.maximum(cnt_scr[...], 1.0)
        out_ref[...] = jnp.dot(pooled, wfc_ref[...], **_DOT) + bfc_ref[...]


def _row_spec(width):
    return pl.BlockSpec((BM, width), lambda i: (i, 0))


def _full_spec(shape):
    nd = len(shape)
    return pl.BlockSpec(shape, lambda i: (0,) * nd)


_tc1 = pl.pallas_call(
    _tc1_body,
    grid=(GRID,),
    in_specs=[pl.BlockSpec((NTILE, BM), lambda i: (0, i)), _row_spec(HID),
              _full_spec((HID, HID))],
    out_specs=[_row_spec(HID), _row_spec(16)],
    out_shape=[jax.ShapeDtypeStruct((N_PAD, HID), jnp.float32),
               jax.ShapeDtypeStruct((N_PAD, 16), jnp.float32)],
)

_tc2 = pl.pallas_call(
    _tc2_body,
    grid=(GRID,),
    in_specs=[_row_spec(HID), _row_spec(HID), _row_spec(16),
              _full_spec((1, HID)), _full_spec((HID, HID))],
    out_specs=_row_spec(HID),
    out_shape=jax.ShapeDtypeStruct((N_PAD, HID), jnp.float32),
)

_tc3 = pl.pallas_call(
    _tc3_body,
    grid=(GRID,),
    in_specs=[_row_spec(HID), _row_spec(HID), _row_spec(16),
              _full_spec((1, HID)), _row_spec(8),
              _full_spec((HID, NC_OUT)), _full_spec((1, NC_OUT))],
    out_specs=_full_spec((NG, NC_OUT)),
    out_shape=jax.ShapeDtypeStruct((NG, NC_OUT), jnp.float32),
    scratch_shapes=[pltpu.VMEM((NG, HID), jnp.float32),
                    pltpu.VMEM((NG, HID), jnp.float32)],
)


@jax.jit
def kernel(x, edge_index, batch, W1, b1, W2, b2, Wfc, bfc):
    pad = E_PAD - E
    src = jnp.concatenate([edge_index[0], jnp.zeros((pad,), jnp.int32)])
    dst = jnp.concatenate(
        [edge_index[1], jnp.full((pad,), N_PAD - 1, jnp.int32)])

    xp = jnp.concatenate([x, jnp.zeros((N_PAD - N, HID), jnp.float32)])
    batch_p = jnp.concatenate(
        [batch, jnp.full((N_PAD - N,), NG, jnp.int32)])

    degp = _deg_kernel(dst)

    g1, dinv = _tc1(degp, xp, W1)

    acc1 = _agg_kernel(src, dst, g1)
    g2 = _tc2(acc1, g1, dinv, b1.reshape(1, HID), W2)

    acc2 = _agg_kernel(src, dst, g2)
    batch8 = jnp.broadcast_to(batch_p[:, None], (N_PAD, 8))
    return _tc3(acc2, g2, dinv, b2.reshape(1, HID),
                batch8, Wfc, bfc.reshape(1, NC_OUT))


# speedup vs baseline: 9.0013x; 1.2493x over previous
import functools

import jax
import jax.numpy as jnp
from jax import lax
from jax.experimental import pallas as pl
from jax.experimental.pallas import tpu as pltpu
from jax.experimental.pallas import tpu_sc as plsc

N = 10000
E = 320000
HID = 128
NC_OUT = 64
NG = 64

NTILE = 16
CHUNK = 128
N_CHUNKS = 158
EPW = N_CHUNKS * CHUNK
E_PAD = NTILE * EPW
N_PAD = 10240
ROWS_PER_TILE = N_PAD // NTILE

BM = 2048
GRID = N_PAD // BM

_mesh = plsc.VectorSubcoreMesh(core_axis_name="c", subcore_axis_name="s",
                               num_cores=1, num_subcores=NTILE)
_sc_params = pltpu.CompilerParams(needs_layout_passes=False)


@functools.partial(
    pl.kernel,
    out_type=jax.ShapeDtypeStruct((NTILE, N_PAD), jnp.float32),
    mesh=_mesh,
    compiler_params=_sc_params,
    scratch_types=[
        pltpu.VMEM((CHUNK,), jnp.int32),
        pltpu.VMEM((N_PAD,), jnp.float32),
    ],
)
def _deg_kernel(dst_hbm, out_hbm, idx_v, hist_v):
    s = lax.axis_index("s")

    zv = jnp.zeros((16,), jnp.float32)
    pl.loop(0, N_PAD // 16)(
        lambda i: hist_v.__setitem__(pl.ds(i * 16, 16), zv))

    ones = jnp.ones((16,), jnp.float32)

    def step(k):
        base = s * EPW + k * CHUNK
        pltpu.sync_copy(dst_hbm.at[pl.ds(base, CHUNK)], idx_v)
        for j in range(CHUNK // 16):
            idx16 = idx_v[pl.ds(j * 16, 16)]
            plsc.addupdate_scatter(hist_v, [idx16], ones)

    pl.loop(0, N_CHUNKS)(step)
    pltpu.sync_copy(hist_v, out_hbm.at[s])


@functools.partial(
    pl.kernel,
    out_type=jax.ShapeDtypeStruct((N_PAD, HID), jnp.float32),
    mesh=_mesh,
    compiler_params=_sc_params,
    scratch_types=[
        pltpu.VMEM((2, CHUNK), jnp.int32),
        pltpu.VMEM((2, CHUNK), jnp.int32),
        pltpu.VMEM((2, CHUNK, HID), jnp.float32),
        pltpu.VMEM_SHARED((N_PAD, HID), jnp.float32),
        pltpu.SemaphoreType.DMA,
        pltpu.SemaphoreType.DMA,
    ],
)
def _agg_kernel(src_hbm, dst_hbm, g_hbm, out_hbm,
                srcs_v, dsts_v, rows_v, acc_sh, sem_a, sem_b):
    s = lax.axis_index("s")

    zv = jnp.zeros((16,), jnp.float32)

    def zrow(i):
        for j in range(HID // 16):
            rows_v[0, i, pl.ds(j * 16, 16)] = zv

    pl.loop(0, CHUNK)(zrow)

    row0 = s * ROWS_PER_TILE
    for r in range(0, ROWS_PER_TILE, CHUNK):
        pltpu.sync_copy(rows_v.at[0],
                        acc_sh.at[pl.ds(row0 + r, CHUNK)])
    plsc.subcore_barrier()

    def step2(kk):
        base = s * EPW + kk * (2 * CHUNK)
        pltpu.sync_copy(src_hbm.at[pl.ds(base, CHUNK)], srcs_v.at[0])
        pltpu.sync_copy(dst_hbm.at[pl.ds(base, CHUNK)], dsts_v.at[0])
        cp_a = pltpu.async_copy(g_hbm.at[srcs_v.at[0]], rows_v.at[0], sem_a)
        pltpu.sync_copy(src_hbm.at[pl.ds(base + CHUNK, CHUNK)], srcs_v.at[1])
        pltpu.sync_copy(dst_hbm.at[pl.ds(base + CHUNK, CHUNK)], dsts_v.at[1])
        cp_b = pltpu.async_copy(g_hbm.at[srcs_v.at[1]], rows_v.at[1], sem_b)
        cp_a.wait()
        pltpu.sync_copy(rows_v.at[0], acc_sh.at[dsts_v.at[0]], add=True)
        cp_b.wait()
        pltpu.sync_copy(rows_v.at[1], acc_sh.at[dsts_v.at[1]], add=True)

    pl.loop(0, N_CHUNKS // 2)(step2)
    plsc.subcore_barrier()

    for r in range(0, ROWS_PER_TILE, CHUNK):
        pltpu.sync_copy(acc_sh.at[pl.ds(row0 + r, CHUNK)], rows_v.at[0])
        pltpu.sync_copy(rows_v.at[0],
                        out_hbm.at[pl.ds(row0 + r, CHUNK)])


_DOT = dict(preferred_element_type=jnp.float32,
            precision=lax.Precision.HIGHEST)


def _tc1_body(degp_ref, x_ref, w1_ref, g_ref, dinv_ref):
    ones16 = jnp.ones((NTILE, 1), jnp.float32)
    deg = lax.dot_general(degp_ref[...], ones16,
                          (((0,), (0,)), ((), ())), **_DOT) + 1.0
    dinv = lax.rsqrt(deg)
    h = jnp.dot(x_ref[...], w1_ref[...], **_DOT)
    g_ref[...] = h * dinv
    dinv_ref[...] = jnp.broadcast_to(dinv, (BM, 16))


def _tc2_body(acc_ref, g1_ref, dinv_ref, b1_ref, w2_ref, g2_ref):
    dinv = dinv_ref[:, 0:1]
    out1 = (acc_ref[...] + g1_ref[...]) * dinv + b1_ref[...]
    out1 = jnp.maximum(out1, 0.0)
    g2_ref[...] = jnp.dot(out1, w2_ref[...], **_DOT) * dinv


def _tc3_body(acc_ref, g2_ref, dinv_ref, b2_ref, batch_ref,
              wfc_ref, bfc_ref, out_ref, sums_scr, cnt_scr):
    i = pl.program_id(0)

    @pl.when(i == 0)
    def _():
        sums_scr[...] = jnp.zeros_like(sums_scr)
        cnt_scr[...] = jnp.zeros_like(cnt_scr)

    dinv = dinv_ref[:, 0:1]
    out2 = (acc_ref[...] + g2_ref[...]) * dinv + b2_ref[...]
    out2 = jnp.maximum(out2, 0.0)

    gids = lax.broadcasted_iota(jnp.int32, (BM, NG), 1)
    onehot = (batch_ref[:, 0:1] == gids).astype(jnp.float32)
    sums_scr[...] += lax.dot_general(
        onehot, out2, (((0,), (0,)), ((), ())), **_DOT)
    cnt = jnp.sum(onehot, axis=0)
    cnt_scr[...] += jnp.broadcast_to(cnt[:, None], (NG, HID))

    @pl.when(i == GRID - 1)
    def _():
        pooled = sums_scr[...] / jnp.maximum(cnt_scr[...], 1.0)
        out_ref[...] = jnp.dot(pooled, wfc_ref[...], **_DOT) + bfc_ref[...]


def _row_spec(width):
    return pl.BlockSpec((BM, width), lambda i: (i, 0))


def _full_spec(shape):
    nd = len(shape)
    return pl.BlockSpec(shape, lambda i: (0,) * nd)


_tc1 = pl.pallas_call(
    _tc1_body,
    grid=(GRID,),
    in_specs=[pl.BlockSpec((NTILE, BM), lambda i: (0, i)), _row_spec(HID),
              _full_spec((HID, HID))],
    out_specs=[_row_spec(HID), _row_spec(16)],
    out_shape=[jax.ShapeDtypeStruct((N_PAD, HID), jnp.float32),
               jax.ShapeDtypeStruct((N_PAD, 16), jnp.float32)],
)

_tc2 = pl.pallas_call(
    _tc2_body,
    grid=(GRID,),
    in_specs=[_row_spec(HID), _row_spec(HID), _row_spec(16),
              _full_spec((1, HID)), _full_spec((HID, HID))],
    out_specs=_row_spec(HID),
    out_shape=jax.ShapeDtypeStruct((N_PAD, HID), jnp.float32),
)

_tc3 = pl.pallas_call(
    _tc3_body,
    grid=(GRID,),
    in_specs=[_row_spec(HID), _row_spec(HID), _row_spec(16),
              _full_spec((1, HID)), _row_spec(8),
              _full_spec((HID, NC_OUT)), _full_spec((1, NC_OUT))],
    out_specs=_full_spec((NG, NC_OUT)),
    out_shape=jax.ShapeDtypeStruct((NG, NC_OUT), jnp.float32),
    scratch_shapes=[pltpu.VMEM((NG, HID), jnp.float32),
                    pltpu.VMEM((NG, HID), jnp.float32)],
)


@jax.jit
def kernel(x, edge_index, batch, W1, b1, W2, b2, Wfc, bfc):
    pad = E_PAD - E
    src = jnp.concatenate([edge_index[0], jnp.zeros((pad,), jnp.int32)])
    dst = jnp.concatenate(
        [edge_index[1], jnp.full((pad,), N_PAD - 1, jnp.int32)])

    xp = jnp.concatenate([x, jnp.zeros((N_PAD - N, HID), jnp.float32)])
    batch_p = jnp.concatenate(
        [batch, jnp.full((N_PAD - N,), NG, jnp.int32)])

    degp = _deg_kernel(dst)

    g1, dinv = _tc1(degp, xp, W1)

    acc1 = _agg_kernel(src, dst, g1)
    g2 = _tc2(acc1, g1, dinv, b1.reshape(1, HID), W2)

    acc2 = _agg_kernel(src, dst, g2)
    batch8 = jnp.broadcast_to(batch_p[:, None], (N_PAD, 8))
    return _tc3(acc2, g2, dinv, b2.reshape(1, HID),
                batch8, Wfc, bfc.reshape(1, NC_OUT))
